# Initial kernel scaffold; baseline (speedup 1.0000x reference)
import jax
import jax.numpy as jnp
from jax.experimental import pallas as pl


def kernel(PlayType, table):
    raise NotImplementedError("write your pallas kernel here")


# speedup vs baseline: 1.1095x; 1.1095x over previous
import functools

import jax
import jax.numpy as jnp
from jax import lax
from jax.experimental import pallas as pl
from jax.experimental.pallas import tpu as pltpu
from jax.experimental.pallas import tpu_sc as plsc

VOCAB = 1000000
EMBED_DIM = 32
BATCH = 16384
HIST = 50

TOTAL = BATCH * HIST
NUM_WORKERS = 32
PER_W = TOTAL // NUM_WORKERS
CHUNK = 3200
NCHUNK = PER_W // CHUNK

assert PER_W * NUM_WORKERS == TOTAL
assert CHUNK * NCHUNK == PER_W

_mesh = plsc.VectorSubcoreMesh(core_axis_name="c", subcore_axis_name="s")


@functools.partial(
    pl.kernel,
    out_type=jax.ShapeDtypeStruct((TOTAL, EMBED_DIM), jnp.float32),
    mesh=_mesh,
    scratch_types=[
        pltpu.VMEM((CHUNK,), jnp.int32),
        pltpu.VMEM((CHUNK, EMBED_DIM), jnp.float32),
        pltpu.SemaphoreType.DMA,
    ],
    compiler_params=pltpu.CompilerParams(use_tc_tiling_on_sc=False),
)
def _sc_gather(idx_hbm, table_hbm, out_hbm, idx_v, rows_v, sem):
    wid = lax.axis_index("s") * 2 + lax.axis_index("c")
    base = wid * PER_W

    def step(g, carry):
        off = pl.multiple_of(base + g * CHUNK, 8)
        pltpu.sync_copy(idx_hbm.at[pl.ds(off, CHUNK)], idx_v)
        pltpu.async_copy(table_hbm.at[idx_v], rows_v, sem).wait()
        pltpu.sync_copy(rows_v, out_hbm.at[pl.ds(off, CHUNK)])
        return carry

    lax.fori_loop(0, NCHUNK, step, 0)


def kernel(PlayType, table):
    flat = PlayType.reshape(-1)
    out = _sc_gather(flat, table)
    return out.reshape(PlayType.shape + (table.shape[1],))


# speedup vs baseline: 1.1129x; 1.0031x over previous
import functools

import jax
import jax.numpy as jnp
from jax import lax
from jax.experimental import pallas as pl
from jax.experimental.pallas import tpu as pltpu
from jax.experimental.pallas import tpu_sc as plsc

VOCAB = 1000000
EMBED_DIM = 32
BATCH = 16384
HIST = 50

TOTAL = BATCH * HIST
NUM_WORKERS = 32
PER_W = TOTAL // NUM_WORKERS
CHUNK = 1600
NCHUNK = PER_W // CHUNK

assert PER_W * NUM_WORKERS == TOTAL
assert CHUNK * NCHUNK == PER_W

_mesh = plsc.VectorSubcoreMesh(core_axis_name="c", subcore_axis_name="s")


@functools.partial(
    pl.kernel,
    out_type=jax.ShapeDtypeStruct((TOTAL, EMBED_DIM), jnp.float32),
    mesh=_mesh,
    scratch_types=[
        pltpu.VMEM((2, CHUNK), jnp.int32),
        pltpu.VMEM((2, CHUNK, EMBED_DIM), jnp.float32),
        pltpu.SemaphoreType.DMA,
        pltpu.SemaphoreType.DMA,
    ],
    compiler_params=pltpu.CompilerParams(use_tc_tiling_on_sc=False),
)
def _sc_gather(idx_hbm, table_hbm, out_hbm, idx_v, rows_v, gsem, osem):
    wid = lax.axis_index("s") * 2 + lax.axis_index("c")
    base = wid * PER_W

    def off(g):
        return pl.multiple_of(base + g * CHUNK, 8)

    def idx_load(g):
        pltpu.sync_copy(idx_hbm.at[pl.ds(off(g), CHUNK)], idx_v.at[g % 2])

    def gather_start(g):
        return pltpu.async_copy(table_hbm.at[idx_v.at[g % 2]],
                                rows_v.at[g % 2], gsem)

    def out_start(g):
        return pltpu.async_copy(rows_v.at[g % 2],
                                out_hbm.at[pl.ds(off(g), CHUNK)], osem)

    gathers = {}
    outs = {}
    idx_load(0)
    gathers[0] = gather_start(0)
    for g in range(NCHUNK):
        if g + 1 < NCHUNK:
            idx_load(g + 1)
            if g >= 1:
                outs[g - 1].wait()
            gathers[g + 1] = gather_start(g + 1)
        gathers[g].wait()
        outs[g] = out_start(g)
    outs[NCHUNK - 2].wait()
    outs[NCHUNK - 1].wait()


def kernel(PlayType, table):
    flat = PlayType.reshape(-1)
    out = _sc_gather(flat, table)
    return out.reshape(PlayType.shape + (table.shape[1],))
